# Initial kernel scaffold; baseline (speedup 1.0000x reference)
import jax
import jax.numpy as jnp
from jax.experimental import pallas as pl


def kernel(image, dfeat, neighbor, W0_0, b0_0, W1_0, b1_0, Wo_0, bo_0, W0_1, b0_1, W1_1, b1_1, Wo_1, bo_1):
    raise NotImplementedError("write your pallas kernel here")


# speedup vs baseline: 4.3486x; 4.3486x over previous
import functools

import jax
import jax.numpy as jnp
from jax.experimental import pallas as pl

NATOMS0 = 128
N_ATOMS = 256
NFEAT = 42
NNEI = 100
BATCH = 8
HID = 15
EXP = NFEAT * 3

ATOMS_PER_STEP = 32
ROWS_PER_STEP = ATOMS_PER_STEP * NNEI
N_STEPS = (BATCH * N_ATOMS) // ATOMS_PER_STEP
STEPS_PER_BATCH = N_ATOMS // ATOMS_PER_STEP


def _energy_body(im_ref,
                 W0a, b0a, W1a, b1a, WoTa, boa, W0Ta, W1Ta,
                 W0b, b0b, W1b, b1b, WoTb, bob, W0Tb, W1Tb,
                 ei_ref, etot_ref, ige_ref):
    x = im_ref[...]
    f_iota = jax.lax.broadcasted_iota(jnp.int32, (NFEAT, EXP), 0)
    k_iota = jax.lax.broadcasted_iota(jnp.int32, (NFEAT, EXP), 1)
    R = (k_iota // 3 == f_iota).astype(jnp.bfloat16)

    def one_type(xt, W0, b0, W1, b1, WoT, bo, W0T, W1T):
        h0 = jnp.dot(xt, W0[...], preferred_element_type=jnp.float32) + b0[...]
        h1 = jnp.dot(h0, W1[...], preferred_element_type=jnp.float32) + b1[...]
        r = jnp.maximum(h1, 0.0)
        e = jnp.sum(r * WoT[...], axis=1) + bo[0, 0]
        g = jnp.where(h1 > 0.0, WoT[...], 0.0)
        dh0 = jnp.dot(g, W1T[...], preferred_element_type=jnp.float32)
        dx = jnp.dot(dh0, W0T[...], preferred_element_type=jnp.float32)
        ige = jnp.dot(dx.astype(jnp.bfloat16), R,
                      preferred_element_type=jnp.float32)
        return e, ige.astype(jnp.bfloat16)

    x0 = x[:, :NATOMS0, :].reshape(BATCH * NATOMS0, NFEAT)
    x1 = x[:, NATOMS0:, :].reshape(BATCH * NATOMS0, NFEAT)
    e0, ige0 = one_type(x0, W0a, b0a, W1a, b1a, WoTa, boa, W0Ta, W1Ta)
    e1, ige1 = one_type(x1, W0b, b0b, W1b, b1b, WoTb, bob, W0Tb, W1Tb)

    e0 = e0.reshape(BATCH, NATOMS0)
    e1 = e1.reshape(BATCH, NATOMS0)
    ei_ref[:, :NATOMS0] = e0
    ei_ref[:, NATOMS0:] = e1
    etot_ref[...] = (jnp.sum(e0, axis=1, keepdims=True)
                     + jnp.sum(e1, axis=1, keepdims=True))
    ige_ref[:, :NATOMS0, :] = ige0.reshape(BATCH, NATOMS0, EXP)
    ige_ref[:, NATOMS0:, :] = ige1.reshape(BATCH, NATOMS0, EXP)


def _force_body(nei_ref, ige_ref, d_ref, out_ref):
    nei = nei_ref[0] - 1
    keep = nei != 0
    idx = jnp.where(nei < 0, nei + N_ATOMS, nei)
    idx = jnp.where(keep, idx, -1)

    lane = jax.lax.broadcasted_iota(
        jnp.int32, (ROWS_PER_STEP, N_ATOMS), 1)
    onehot = (idx == lane).astype(jnp.bfloat16)

    gathE = jax.lax.dot_general(
        onehot, ige_ref[0],
        (((1,), (0,)), ((), ())),
        preferred_element_type=jnp.float32)

    P = (gathE * d_ref[...]).astype(jnp.bfloat16)

    a_row = jax.lax.broadcasted_iota(
        jnp.int32, (ATOMS_PER_STEP, ROWS_PER_STEP), 1) // NNEI
    a_atom = jax.lax.broadcasted_iota(
        jnp.int32, (ATOMS_PER_STEP, ROWS_PER_STEP), 0)
    A = (a_row == a_atom).astype(jnp.bfloat16)

    colsum = jax.lax.dot_general(
        A, P, (((1,), (0,)), ((), ())),
        preferred_element_type=jnp.float32)

    s_k = jax.lax.broadcasted_iota(jnp.int32, (EXP, 3), 0)
    s_c = jax.lax.broadcasted_iota(jnp.int32, (EXP, 3), 1)
    S = (s_k % 3 == s_c).astype(jnp.bfloat16)

    out_ref[...] = jax.lax.dot_general(
        colsum.astype(jnp.bfloat16), S,
        (((1,), (0,)), ((), ())),
        preferred_element_type=jnp.float32)


@functools.partial(jax.jit, static_argnames=("interpret",))
def _run(image, dfeat, neighbor,
         W0_0, b0_0, W1_0, b1_0, Wo_0, bo_0,
         W0_1, b0_1, W1_1, b1_1, Wo_1, bo_1,
         interpret=False):
    B, N = image.shape[0], image.shape[1]

    def r2(b):
        return b.reshape(1, -1)

    ei2d, etot, ige = pl.pallas_call(
        _energy_body,
        out_shape=(
            jax.ShapeDtypeStruct((B, N), jnp.float32),
            jax.ShapeDtypeStruct((B, 1), jnp.float32),
            jax.ShapeDtypeStruct((B, N, EXP), jnp.bfloat16),
        ),
        interpret=interpret,
    )(image,
      W0_0, r2(b0_0), W1_0, r2(b1_0), Wo_0.T, bo_0.reshape(1, 1),
      W0_0.T, W1_0.T,
      W0_1, r2(b0_1), W1_1, r2(b1_1), Wo_1.T, bo_1.reshape(1, 1),
      W0_1.T, W1_1.T)

    nei3 = neighbor.reshape(N_STEPS, ROWS_PER_STEP, 1)
    dfeat2 = dfeat.reshape(B * N * NNEI, EXP)

    force = pl.pallas_call(
        _force_body,
        grid=(N_STEPS,),
        in_specs=[
            pl.BlockSpec((1, ROWS_PER_STEP, 1), lambda s: (s, 0, 0)),
            pl.BlockSpec((1, N, EXP), lambda s: (s // STEPS_PER_BATCH, 0, 0)),
            pl.BlockSpec((ROWS_PER_STEP, EXP), lambda s: (s, 0)),
        ],
        out_specs=pl.BlockSpec((ATOMS_PER_STEP, 3), lambda s: (s, 0)),
        out_shape=jax.ShapeDtypeStruct((B * N, 3), jnp.float32),
        interpret=interpret,
    )(nei3, ige, dfeat2)

    return force.reshape(B, N, 3), etot, ei2d[..., None]


def kernel(image, dfeat, neighbor, W0_0, b0_0, W1_0, b1_0, Wo_0, bo_0,
           W0_1, b0_1, W1_1, b1_1, Wo_1, bo_1):
    return _run(image, dfeat, neighbor, W0_0, b0_0, W1_0, b1_0, Wo_0, bo_0,
                W0_1, b0_1, W1_1, b1_1, Wo_1, bo_1)


# speedup vs baseline: 8.6561x; 1.9906x over previous
import functools

import jax
import jax.numpy as jnp
from jax.experimental import pallas as pl
from jax.experimental.pallas import tpu as pltpu

NATOMS0 = 128
N_ATOMS = 256
NFEAT = 42
NNEI = 100
BATCH = 8
HID = 15
ROWS = BATCH * N_ATOMS


def _energy_body(im_ref,
                 W0a, b0a, W1a, b1a, WoTa, boa, W0Ta, W1Ta,
                 W0b, b0b, W1b, b1b, WoTb, bob, W0Tb, W1Tb,
                 ei_ref, etot_ref, ig_ref):
    x = im_ref[...]

    def one_type(xt, W0, b0, W1, b1, WoT, bo, W0T, W1T):
        h0 = jnp.dot(xt, W0[...], preferred_element_type=jnp.float32) + b0[...]
        h1 = jnp.dot(h0, W1[...], preferred_element_type=jnp.float32) + b1[...]
        r = jnp.maximum(h1, 0.0)
        e = jnp.sum(r * WoT[...], axis=1) + bo[0, 0]
        g = jnp.where(h1 > 0.0, WoT[...], 0.0)
        dh0 = jnp.dot(g, W1T[...], preferred_element_type=jnp.float32)
        dx = jnp.dot(dh0, W0T[...], preferred_element_type=jnp.float32)
        return e, dx

    x0 = x[:, :NATOMS0, :].reshape(BATCH * NATOMS0, NFEAT)
    x1 = x[:, NATOMS0:, :].reshape(BATCH * NATOMS0, NFEAT)
    e0, ig0 = one_type(x0, W0a, b0a, W1a, b1a, WoTa, boa, W0Ta, W1Ta)
    e1, ig1 = one_type(x1, W0b, b0b, W1b, b1b, WoTb, bob, W0Tb, W1Tb)

    e0 = e0.reshape(BATCH, NATOMS0)
    e1 = e1.reshape(BATCH, NATOMS0)
    ei_ref[:, :NATOMS0] = e0
    ei_ref[:, NATOMS0:] = e1
    etot_ref[...] = (jnp.sum(e0, axis=1, keepdims=True)
                     + jnp.sum(e1, axis=1, keepdims=True))
    ig_ref[:, :NATOMS0, :] = ig0.reshape(BATCH, NATOMS0, NFEAT)
    ig_ref[:, NATOMS0:, :] = ig1.reshape(BATCH, NATOMS0, NFEAT)


def _force_body(igt_ref, d_ref, nei_ref, out_ref, idx_s, wlo_s, whi_s, acc):
    f = pl.program_id(1)
    step = pl.program_id(0) * NFEAT + f

    @pl.when(step == 0)
    def _prep():
        nei = nei_ref[...] - 1
        keep = nei != 0
        idx = jnp.where(nei < 0, nei + N_ATOMS, nei)
        hi = (idx >= 128) & keep
        lo = (idx < 128) & keep
        idx_s[...] = idx & 127
        wlo_s[...] = lo.astype(jnp.float32)
        whi_s[...] = hi.astype(jnp.float32)

    table = igt_ref[0]
    idx = idx_s[...]
    half = N_ATOMS // 2

    def gather_half(tab_half):
        x = jnp.broadcast_to(tab_half[:, None, :], (BATCH, N_ATOMS, half))
        x = x.reshape(ROWS, half)
        return jnp.take_along_axis(x, idx, axis=1, mode="promise_in_bounds")

    glo = gather_half(table[:, :half])
    ghi = gather_half(table[:, half:])
    contrib = (glo * wlo_s[...] + ghi * whi_s[...]) * d_ref[0]

    @pl.when(f == 0)
    def _init():
        acc[...] = contrib

    @pl.when(f > 0)
    def _acc():
        acc[...] += contrib

    @pl.when(f == NFEAT - 1)
    def _out():
        out_ref[0] = jnp.sum(acc[...], axis=1, keepdims=True)


@functools.partial(jax.jit, static_argnames=("interpret",))
def _run(image, dfeat, neighbor,
         W0_0, b0_0, W1_0, b1_0, Wo_0, bo_0,
         W0_1, b0_1, W1_1, b1_1, Wo_1, bo_1, interpret=False):
    B, N = image.shape[0], image.shape[1]

    def r2(b):
        return b.reshape(1, -1)

    ei2d, etot, ig = pl.pallas_call(
        _energy_body,
        interpret=interpret,
        out_shape=(
            jax.ShapeDtypeStruct((B, N), jnp.float32),
            jax.ShapeDtypeStruct((B, 1), jnp.float32),
            jax.ShapeDtypeStruct((B, N, NFEAT), jnp.float32),
        ),
    )(image,
      W0_0, r2(b0_0), W1_0, r2(b1_0), Wo_0.T, bo_0.reshape(1, 1),
      W0_0.T, W1_0.T,
      W0_1, r2(b0_1), W1_1, r2(b1_1), Wo_1.T, bo_1.reshape(1, 1),
      W0_1.T, W1_1.T)

    igt = jnp.transpose(ig, (2, 0, 1))
    dfeatT = dfeat.transpose(3, 4, 0, 1, 2).reshape(NFEAT * 3, ROWS, NNEI)
    nei2 = neighbor.reshape(ROWS, NNEI)

    out = pl.pallas_call(
        _force_body,
        grid=(3, NFEAT),
        in_specs=[
            pl.BlockSpec((1, B, N), lambda c, f: (f, 0, 0)),
            pl.BlockSpec((1, ROWS, NNEI), lambda c, f: (f * 3 + c, 0, 0)),
            pl.BlockSpec((ROWS, NNEI), lambda c, f: (0, 0)),
        ],
        out_specs=pl.BlockSpec((1, ROWS, 1), lambda c, f: (c, 0, 0)),
        out_shape=jax.ShapeDtypeStruct((3, ROWS, 1), jnp.float32),
        scratch_shapes=[
            pltpu.VMEM((ROWS, NNEI), jnp.int32),
            pltpu.VMEM((ROWS, NNEI), jnp.float32),
            pltpu.VMEM((ROWS, NNEI), jnp.float32),
            pltpu.VMEM((ROWS, NNEI), jnp.float32),
        ],
        interpret=interpret,
    )(igt, dfeatT, nei2)

    force = jnp.transpose(out.reshape(3, B, N), (1, 2, 0))
    return force, etot, ei2d[..., None]


def kernel(image, dfeat, neighbor, W0_0, b0_0, W1_0, b1_0, Wo_0, bo_0,
           W0_1, b0_1, W1_1, b1_1, Wo_1, bo_1):
    return _run(image, dfeat, neighbor, W0_0, b0_0, W1_0, b1_0, Wo_0, bo_0,
                W0_1, b0_1, W1_1, b1_1, Wo_1, bo_1)
